# baseline (speedup 1.0000x reference)
import functools

import jax
import jax.numpy as jnp
from jax import lax
from jax.experimental import pallas as pl
from jax.experimental.pallas import tpu as pltpu
from jax.experimental.pallas import tpu_sc as plsc

_N = 16384
_K = 128
_LANES = 16
_NUM_CORES = 2
_NUM_SUBCORES = 16
_NW = _NUM_CORES * _NUM_SUBCORES
_PER_W = _N // _NW


@functools.partial(
    pl.kernel,
    mesh=plsc.VectorSubcoreMesh(core_axis_name="c", subcore_axis_name="s"),
    out_type=jax.ShapeDtypeStruct((_N,), jnp.int32),
    compiler_params=pltpu.CompilerParams(needs_layout_passes=False),
    scratch_types=[
        pltpu.VMEM((_PER_W,), jnp.int32),
        pltpu.VMEM((_K,), jnp.int32),
        pltpu.VMEM((_K,), jnp.int32),
        pltpu.VMEM((_PER_W,), jnp.int32),
    ],
)
def _encode(x_hbm, cond_hbm, out_hbm, x_v, cond_v, table_v, out_v):
    wid = lax.axis_index("s") * _NUM_CORES + lax.axis_index("c")
    base = wid * _PER_W
    pltpu.sync_copy(x_hbm.at[pl.ds(base, _PER_W)], x_v)
    pltpu.sync_copy(cond_hbm, cond_v)
    for kb in range(_K // _LANES):
        vals = cond_v[pl.ds(kb * _LANES, _LANES)]
        ks = lax.iota(jnp.int32, _LANES) + (kb * _LANES)
        plsc.store_scatter(table_v, [vals], ks)
    for b in range(_PER_W // _LANES):
        xv = x_v[pl.ds(b * _LANES, _LANES)]
        out_v[pl.ds(b * _LANES, _LANES)] = plsc.load_gather(table_v, [xv])
    pltpu.sync_copy(out_v, out_hbm.at[pl.ds(base, _PER_W)])


def kernel(x, condition_tensors):
    idx = _encode(x, condition_tensors)
    return idx.reshape(-1, 1, 1).astype(jnp.int64)


# speedup vs baseline: 1.0923x; 1.0923x over previous
import functools

import jax
import jax.numpy as jnp
from jax import lax
from jax.experimental import pallas as pl
from jax.experimental.pallas import tpu as pltpu
from jax.experimental.pallas import tpu_sc as plsc

_N = 16384
_K = 128
_LANES = 16
_NUM_CORES = 1
_NUM_SUBCORES = 16
_NW = _NUM_CORES * _NUM_SUBCORES
_PER_W = _N // _NW


@functools.partial(
    pl.kernel,
    mesh=plsc.VectorSubcoreMesh(core_axis_name="c", subcore_axis_name="s",
                                num_cores=_NUM_CORES),
    out_type=jax.ShapeDtypeStruct((_N,), jnp.int32),
    compiler_params=pltpu.CompilerParams(needs_layout_passes=False),
    scratch_types=[
        pltpu.VMEM((_PER_W,), jnp.int32),
        pltpu.VMEM((_K,), jnp.int32),
        pltpu.VMEM((_K,), jnp.int32),
        pltpu.VMEM((_PER_W,), jnp.int32),
        pltpu.SemaphoreType.DMA,
        pltpu.SemaphoreType.DMA,
    ],
)
def _encode(x_hbm, cond_hbm, out_hbm, x_v, cond_v, table_v, out_v,
            sem_x, sem_c):
    wid = lax.axis_index("s")
    base = wid * _PER_W
    cp_x = pltpu.async_copy(x_hbm.at[pl.ds(base, _PER_W)], x_v, sem_x)
    cp_c = pltpu.async_copy(cond_hbm, cond_v, sem_c)
    cp_c.wait()
    for kb in range(_K // _LANES):
        vals = cond_v[pl.ds(kb * _LANES, _LANES)]
        ks = lax.iota(jnp.int32, _LANES) + (kb * _LANES)
        plsc.store_scatter(table_v, [vals], ks)
    cp_x.wait()
    for b in range(_PER_W // _LANES):
        xv = x_v[pl.ds(b * _LANES, _LANES)]
        out_v[pl.ds(b * _LANES, _LANES)] = plsc.load_gather(table_v, [xv])
    pltpu.sync_copy(out_v, out_hbm.at[pl.ds(base, _PER_W)])


def kernel(x, condition_tensors):
    idx = _encode(x, condition_tensors)
    return idx.reshape(-1, 1, 1).astype(jnp.int64)
